# Initial kernel scaffold; baseline (speedup 1.0000x reference)
import jax
import jax.numpy as jnp
from jax.experimental import pallas as pl


def kernel(x, window_w, window_b, window_gamma, offset_w, offset_b, offset_gamma, kernel_w, kernel_b, kernel_gamma, v_w, v_b, se_fc1_w, se_fc2_w, out_w):
    raise NotImplementedError("write your pallas kernel here")


# speedup vs baseline: 8.1300x; 8.1300x over previous
import functools

import jax
import jax.numpy as jnp
from jax.experimental import pallas as pl
from jax.experimental.pallas import tpu as pltpu

MIN_WINDOW = 1.0
SCALE_POWER = 0.3


def _rms(z, g, n):
    var = jnp.sum(z * z, axis=-1, keepdims=True) / n
    return z * jax.lax.rsqrt(var + 1e-6) * g


def _proj_kernel(x_ref, w_ref, b_ref, v_ref, pre_ref, *, Lb, C, H, D):
    full = jnp.dot(x_ref[0], w_ref[...],
                   preferred_element_type=jnp.float32) + b_ref[0]
    v_ref[0] = full[:, :C].reshape(Lb, H, D)
    pre_ref[0] = full[:, C:]


def _coef_kernel(prek_ref, prewo_ref, kg_ref, wg_ref, og_ref,
                 c_ref, ws_ref, *, Lb, L, H, K, HK, MW, MO, HWM, MAXD, R):
    i = pl.program_id(1)
    kpre = prek_ref[0]
    wo = prewo_ref[0]
    wp = wo[:, :128]
    op = wo[:, 128:256]

    kw = jax.nn.silu(_rms(kpre, kg_ref[0], HK)).reshape(Lb, H, K)

    wvar = jnp.sum(wp * wp, axis=-1, keepdims=True) / H
    wn = wp[:, :H] * jax.lax.rsqrt(wvar + 1e-6) * wg_ref[0]
    sizes = MIN_WINDOW + jax.nn.sigmoid(wn) * (MW - MIN_WINDOW)
    hw = jnp.maximum(sizes * 0.5, 0.5)

    ovar = jnp.sum(op * op, axis=-1, keepdims=True) / H
    on = op[:, :H] * jax.lax.rsqrt(ovar + 1e-6) * og_ref[0]
    off = jnp.tanh(on) * MO

    lpos = (i * Lb + jax.lax.broadcasted_iota(jnp.int32, (Lb, H), 0)
            ).astype(jnp.float32)
    kio = jax.lax.broadcasted_iota(jnp.int32, (Lb, H, K), 2).astype(jnp.float32)

    wt_abs = []
    for a_abs in range(HWM + 1):
        a = a_abs / hw
        wwt = jnp.exp(-a * a)
        npos = jnp.minimum(a, 1.0) * (K - 1)
        hat = jnp.maximum(1.0 - jnp.abs(npos[..., None] - kio), 0.0)
        kwt = jnp.sum(kw * hat, axis=-1)
        wt_abs.append((jnp.maximum(kwt, 0.0) + 1.0) * wwt)

    rio = jax.lax.broadcasted_iota(jnp.int32, (Lb, H, R), 2
                                   ).astype(jnp.float32) - MAXD
    c = jnp.zeros((Lb, H, R), jnp.float32)
    ws = jnp.zeros((Lb, H), jnp.float32)
    for s in range(-HWM, HWM + 1):
        posn = lpos + off + float(s)
        valid = ((posn >= 0) & (posn < L)).astype(jnp.float32)
        wv = wt_abs[abs(s)] * valid
        ws = ws + wv
        pc = jnp.clip(posn, 0.0, L - 1.001)
        pr = pc - lpos
        c = c + wv[..., None] * jnp.maximum(1.0 - jnp.abs(pr[..., None] - rio), 0.0)

    c_ref[0] = c
    ws_ref[0] = ws


def _band_kernel(c_ref, ws_ref, vp_ref, vc_ref, vn_ref,
                 mid_ref, sums_ref, scr, *, Lb, H, D, MAXD, R):
    i = pl.program_id(1)
    scr[0:Lb] = vp_ref[0]
    scr[Lb:2 * Lb] = vc_ref[0]
    scr[2 * Lb:3 * Lb] = vn_ref[0]
    cb = c_ref[0]
    acc = jnp.zeros((Lb, H, D), jnp.float32)
    for r in range(R):
        acc = acc + cb[:, :, r][..., None] * scr[Lb - MAXD + r: 2 * Lb - MAXD + r]
    mid = (acc / jnp.maximum(ws_ref[0], 1.0)[..., None]).reshape(Lb, H * D)
    mid_ref[0] = mid
    colsum = jnp.sum(mid, axis=0, keepdims=True)

    @pl.when(i == 0)
    def _():
        sums_ref[0] = colsum

    @pl.when(i > 0)
    def _():
        sums_ref[0] = sums_ref[0] + colsum


def _se_kernel(sums_ref, f1_ref, f2_ref, scale_ref, *, L):
    mean = sums_ref[:, 0, :] / L
    h1 = jax.nn.silu(jnp.dot(mean, f1_ref[...], preferred_element_type=jnp.float32))
    scale_ref[:, 0, :] = jax.nn.sigmoid(
        jnp.dot(h1, f2_ref[...], preferred_element_type=jnp.float32))


def _out_kernel(mid_ref, scale_ref, w_ref, out_ref):
    y = jnp.dot(mid_ref[0] * scale_ref[0],
                w_ref[...], preferred_element_type=jnp.float32)
    out_ref[0] = jax.nn.silu(y)


def kernel(x, window_w, window_b, window_gamma, offset_w, offset_b, offset_gamma,
           kernel_w, kernel_b, kernel_gamma, v_w, v_b, se_fc1_w, se_fc2_w, out_w):
    B, L, C = x.shape
    H = window_w.shape[0]
    HK = kernel_w.shape[0]
    K = HK // H
    D = C // H
    MW = min(int(L ** SCALE_POWER), K)
    HWM = MW // 2
    MO = int(L ** SCALE_POWER)
    MAXD = HWM + MO
    R = 2 * MAXD + 2

    pad = jnp.zeros((128 - H, C), jnp.float32)
    Wcat = jnp.concatenate(
        [v_w, kernel_w, window_w, pad, offset_w, pad], axis=0).T
    F = C + HK + 256
    F2 = HK + 256
    bpad = jnp.zeros((128 - H,), jnp.float32)
    bcat = jnp.concatenate(
        [v_b, kernel_b, window_b, bpad, offset_b, bpad]).reshape(1, F)

    LbA = 256
    NA = L // LbA
    v3, pre2 = pl.pallas_call(
        functools.partial(_proj_kernel, Lb=LbA, C=C, H=H, D=D),
        grid=(B, NA),
        in_specs=[
            pl.BlockSpec((1, LbA, C), lambda b, i: (b, i, 0)),
            pl.BlockSpec((C, F), lambda b, i: (0, 0)),
            pl.BlockSpec((1, F), lambda b, i: (0, 0)),
        ],
        out_specs=[
            pl.BlockSpec((1, LbA, H, D), lambda b, i: (b, i, 0, 0)),
            pl.BlockSpec((1, LbA, F2), lambda b, i: (b, i, 0)),
        ],
        out_shape=[
            jax.ShapeDtypeStruct((B, L, H, D), jnp.float32),
            jax.ShapeDtypeStruct((B, L, F2), jnp.float32),
        ],
    )(x, Wcat, bcat)

    LbB = 128
    NB = L // LbB
    c, ws = pl.pallas_call(
        functools.partial(_coef_kernel, Lb=LbB, L=L, H=H, K=K, HK=HK,
                          MW=MW, MO=MO, HWM=HWM, MAXD=MAXD, R=R),
        grid=(B, NB),
        in_specs=[
            pl.BlockSpec((1, LbB, HK), lambda b, i: (b, i, 0)),
            pl.BlockSpec((1, LbB, 256), lambda b, i: (b, i, HK // 256)),
            pl.BlockSpec((1, HK), lambda b, i: (0, 0)),
            pl.BlockSpec((1, H), lambda b, i: (0, 0)),
            pl.BlockSpec((1, H), lambda b, i: (0, 0)),
        ],
        out_specs=[
            pl.BlockSpec((1, LbB, H, R), lambda b, i: (b, i, 0, 0)),
            pl.BlockSpec((1, LbB, H), lambda b, i: (b, i, 0)),
        ],
        out_shape=[
            jax.ShapeDtypeStruct((B, L, H, R), jnp.float32),
            jax.ShapeDtypeStruct((B, L, H), jnp.float32),
        ],
    )(pre2, pre2, kernel_gamma.reshape(1, HK),
      window_gamma.reshape(1, H), offset_gamma.reshape(1, H))

    Lb = 128
    NL = L // Lb
    mid, sums = pl.pallas_call(
        functools.partial(_band_kernel, Lb=Lb, H=H, D=D, MAXD=MAXD, R=R),
        grid=(B, NL),
        in_specs=[
            pl.BlockSpec((1, Lb, H, R), lambda b, i: (b, i, 0, 0)),
            pl.BlockSpec((1, Lb, H), lambda b, i: (b, i, 0)),
            pl.BlockSpec((1, Lb, H, D),
                         lambda b, i: (b, jnp.maximum(i - 1, 0), 0, 0)),
            pl.BlockSpec((1, Lb, H, D), lambda b, i: (b, i, 0, 0)),
            pl.BlockSpec((1, Lb, H, D),
                         lambda b, i, NL=NL: (b, jnp.minimum(i + 1, NL - 1), 0, 0)),
        ],
        out_specs=[
            pl.BlockSpec((1, Lb, C), lambda b, i: (b, i, 0)),
            pl.BlockSpec((1, 1, C), lambda b, i: (b, 0, 0)),
        ],
        out_shape=[
            jax.ShapeDtypeStruct((B, L, C), jnp.float32),
            jax.ShapeDtypeStruct((B, 1, C), jnp.float32),
        ],
        scratch_shapes=[pltpu.VMEM((3 * Lb, H, D), jnp.float32)],
    )(c, ws, v3, v3, v3)

    scale = pl.pallas_call(
        functools.partial(_se_kernel, L=L),
        in_specs=[
            pl.BlockSpec(sums.shape, lambda: (0, 0, 0)),
            pl.BlockSpec(se_fc1_w.T.shape, lambda: (0, 0)),
            pl.BlockSpec(se_fc2_w.T.shape, lambda: (0, 0)),
        ],
        out_specs=pl.BlockSpec((B, 1, C), lambda: (0, 0, 0)),
        out_shape=jax.ShapeDtypeStruct((B, 1, C), jnp.float32),
    )(sums, se_fc1_w.T, se_fc2_w.T)

    out = pl.pallas_call(
        _out_kernel,
        grid=(B, NL),
        in_specs=[
            pl.BlockSpec((1, Lb, C), lambda b, i: (b, i, 0)),
            pl.BlockSpec((1, 1, C), lambda b, i: (b, 0, 0)),
            pl.BlockSpec((C, C), lambda b, i: (0, 0)),
        ],
        out_specs=pl.BlockSpec((1, Lb, C), lambda b, i: (b, i, 0)),
        out_shape=jax.ShapeDtypeStruct((B, L, C), jnp.float32),
    )(mid, scale, out_w.T)

    return out
